# baseline (speedup 1.0000x reference)
import functools

import jax
import jax.numpy as jnp
from jax import lax
from jax.experimental import pallas as pl
from jax.experimental.pallas import tpu as pltpu
from jax.experimental.pallas import tpu_sc as plsc

N_NODES = 10000
D = 128
N_EDGES = 320000
EPS = 1e-5

_R = 2000
_G = N_NODES // _R

_NC = 2
_NS = 16
_CH = 128
_CPT = 80
_NCHUNK = _CPT * _NC * _NS
_EPAD = _NCHUNK * _CH
_NPAD = 10240
_RPT = _NPAD // _NS
_ZR = 128


def _fc1_body(x_ref, w_ref, b_ref, h0_ref, st_ref):
    i = pl.program_id(0)
    h0 = jnp.dot(x_ref[...], w_ref[...], preferred_element_type=jnp.float32)
    h0 = h0 + b_ref[...]
    h0_ref[...] = h0

    @pl.when(i == 0)
    def _():
        st_ref[...] = jnp.zeros_like(st_ref)

    st_ref[0:1, :] += jnp.sum(h0, axis=0, keepdims=True)
    st_ref[1:2, :] += jnp.sum(h0 * h0, axis=0, keepdims=True)


def _proj_body(h0_ref, st_ref, g_ref, be_ref, wr_ref, wn_ref, bgc_ref,
               hr_ref, m_ref):
    st = st_ref[...]
    mean = st[0:1, :] * (1.0 / N_NODES)
    var = st[1:2, :] * (1.0 / N_NODES) - mean * mean
    a = g_ref[...] * lax.rsqrt(var + EPS)
    c = be_ref[...] - mean * a
    h = h0_ref[...] * a + c
    hr_ref[...] = jnp.dot(h, wr_ref[...],
                          preferred_element_type=jnp.float32) + bgc_ref[...]
    m_ref[...] = jnp.dot(h, wn_ref[...], preferred_element_type=jnp.float32)


def _gc_body(hr_ref, a0_ref, a1_ref, w2_ref, b2_ref, t_ref, st_ref):
    i = pl.program_id(0)
    gc = hr_ref[...] + a0_ref[0] + a1_ref[0]
    g = gc * 0.5 * (1.0 + lax.erf(gc * 0.7071067811865476))
    t = jnp.dot(g, w2_ref[...], preferred_element_type=jnp.float32) + b2_ref[...]
    t_ref[...] = t

    @pl.when(i == 0)
    def _():
        st_ref[...] = jnp.zeros_like(st_ref)

    st_ref[0:1, :] += jnp.sum(t, axis=0, keepdims=True)
    st_ref[1:2, :] += jnp.sum(t * t, axis=0, keepdims=True)


def _fin_body(t_ref, st_ref, g_ref, be_ref, x_ref, o_ref):
    st = st_ref[...]
    mean = st[0:1, :] * (1.0 / N_NODES)
    var = st[1:2, :] * (1.0 / N_NODES) - mean * mean
    a = g_ref[...] * lax.rsqrt(var + EPS)
    c = be_ref[...] - mean * a
    o_ref[...] = t_ref[...] * a + c + x_ref[...]


_row_spec = pl.BlockSpec((_R, D), lambda i: (i, 0))
_full_spec = pl.BlockSpec((D, D), lambda i: (0, 0))
_vec_spec = pl.BlockSpec((1, D), lambda i: (0, 0))
_st_spec = pl.BlockSpec((8, D), lambda i: (0, 0))
_rows_out = jax.ShapeDtypeStruct((N_NODES, D), jnp.float32)
_st_out = jax.ShapeDtypeStruct((8, D), jnp.float32)


_fc1 = pl.pallas_call(
    _fc1_body, grid=(_G,),
    in_specs=[_row_spec, _full_spec, _vec_spec],
    out_specs=[_row_spec, _st_spec],
    out_shape=[_rows_out, _st_out],
)

_proj = pl.pallas_call(
    _proj_body, grid=(_G,),
    in_specs=[_row_spec, _st_spec, _vec_spec, _vec_spec, _full_spec,
              _full_spec, _vec_spec],
    out_specs=[_row_spec, _row_spec],
    out_shape=[_rows_out, _rows_out],
)

_gc = pl.pallas_call(
    _gc_body, grid=(_G,),
    in_specs=[_row_spec,
              pl.BlockSpec((1, _R, D), lambda i: (0, i, 0)),
              pl.BlockSpec((1, _R, D), lambda i: (1, i, 0)),
              _full_spec, _vec_spec],
    out_specs=[_row_spec, _st_spec],
    out_shape=[_rows_out, _st_out],
)

_fin = pl.pallas_call(
    _fin_body, grid=(_G,),
    in_specs=[_row_spec, _st_spec, _vec_spec, _vec_spec, _row_spec],
    out_specs=_row_spec,
    out_shape=_rows_out,
)


def _sc_body(m_hbm, src_hbm, dst_hbm, out_hbm,
             sidx0, sidx1, didx0, didx1, rows0, rows1, agg_sh, gsem0, gsem1):
    cid = lax.axis_index("c")
    sid = lax.axis_index("s")
    wid = cid * _NS + sid
    e0 = wid * (_CPT * _CH)
    row0 = sid * _RPT

    z16 = jnp.zeros((16,), jnp.float32)

    def _zrow(r, carry):
        for j in range(D // 16):
            rows0[r, pl.ds(j * 16, 16)] = z16
        return carry

    lax.fori_loop(0, _ZR, _zrow, 0)

    def _zcp(t, carry):
        pltpu.sync_copy(rows0, agg_sh.at[pl.ds(row0 + t * _ZR, _ZR)])
        return carry

    lax.fori_loop(0, _RPT // _ZR, _zcp, 0)
    plsc.subcore_barrier()

    def _step(c, carry):
        eoff = e0 + c * _CH
        pltpu.sync_copy(src_hbm.at[pl.ds(eoff, _CH)], sidx0)
        pltpu.sync_copy(dst_hbm.at[pl.ds(eoff, _CH)], didx0)
        pltpu.async_copy(m_hbm.at[sidx0], rows0, gsem0).wait()
        pltpu.sync_copy(rows0, agg_sh.at[didx0], add=True)
        return carry

    lax.fori_loop(0, _CPT, _step, 0)

    plsc.subcore_barrier()

    def _wout(t, carry):
        r0 = row0 + t * _ZR
        pltpu.sync_copy(agg_sh.at[pl.ds(r0, _ZR)],
                        out_hbm.at[cid, pl.ds(r0, _ZR)])
        return carry

    lax.fori_loop(0, _RPT // _ZR, _wout, 0)


@functools.cache
def _make_segsum():
    return functools.partial(
        pl.kernel,
        mesh=plsc.VectorSubcoreMesh(core_axis_name="c", subcore_axis_name="s"),
        out_type=jax.ShapeDtypeStruct((_NC, _NPAD, D), jnp.float32),
        scratch_types=[
            pltpu.VMEM((_CH,), jnp.int32),
            pltpu.VMEM((_CH,), jnp.int32),
            pltpu.VMEM((_CH,), jnp.int32),
            pltpu.VMEM((_CH,), jnp.int32),
            pltpu.VMEM((_CH, D), jnp.float32),
            pltpu.VMEM((_CH, D), jnp.float32),
            pltpu.VMEM_SHARED((_NPAD, D), jnp.float32),
            pltpu.SemaphoreType.DMA,
            pltpu.SemaphoreType.DMA,
        ],
    )(_sc_body)


def kernel(x, edge_index, W1, b1, g1, be1, Wr, Wn, bgc, W2, b2, g2, be2):
    ei = edge_index.astype(jnp.int32)
    npad = _EPAD - N_EDGES
    src = jnp.concatenate([ei[0], jnp.zeros((npad,), jnp.int32)])
    pad_dst = N_NODES + jnp.arange(npad, dtype=jnp.int32) % (_NPAD - N_NODES)
    dst = jnp.concatenate([ei[1], pad_dst])
    b1r = b1.reshape(1, D)
    g1r = g1.reshape(1, D)
    be1r = be1.reshape(1, D)
    bgcr = bgc.reshape(1, D)
    b2r = b2.reshape(1, D)
    g2r = g2.reshape(1, D)
    be2r = be2.reshape(1, D)

    h0, st1 = _fc1(x, W1, b1r)
    hr, m = _proj(h0, st1, g1r, be1r, Wr, Wn, bgcr)
    aggp = _make_segsum()(m, src, dst)
    t, st2 = _gc(hr, aggp, aggp, W2, b2r)
    return _fin(t, st2, g2r, be2r, x)


# speedup vs baseline: 2.1511x; 2.1511x over previous
import functools

import jax
import jax.numpy as jnp
from jax import lax
from jax.experimental import pallas as pl
from jax.experimental.pallas import tpu as pltpu
from jax.experimental.pallas import tpu_sc as plsc

N_NODES = 10000
D = 128
N_EDGES = 320000
EPS = 1e-5

_R = 2000
_G = N_NODES // _R

_NC = 2
_NS = 16
_E_PER_TILE = N_EDGES // (_NC * _NS)
_CH = 128
_NFULL = _E_PER_TILE // _CH
_TAIL = _E_PER_TILE - _NFULL * _CH
_NPAD = 10240
_RPT = _NPAD // _NS
_ZR = 128


def _fc1_body(x_ref, w_ref, b_ref, h0_ref, st_ref):
    i = pl.program_id(0)
    h0 = jnp.dot(x_ref[...], w_ref[...], preferred_element_type=jnp.float32)
    h0 = h0 + b_ref[...]
    h0_ref[...] = h0

    @pl.when(i == 0)
    def _():
        st_ref[...] = jnp.zeros_like(st_ref)

    st_ref[0:1, :] += jnp.sum(h0, axis=0, keepdims=True)
    st_ref[1:2, :] += jnp.sum(h0 * h0, axis=0, keepdims=True)


def _proj_body(h0_ref, st_ref, g_ref, be_ref, wr_ref, wn_ref, bgc_ref,
               hr_ref, m_ref):
    st = st_ref[...]
    mean = st[0:1, :] * (1.0 / N_NODES)
    var = st[1:2, :] * (1.0 / N_NODES) - mean * mean
    a = g_ref[...] * lax.rsqrt(var + EPS)
    c = be_ref[...] - mean * a
    h = h0_ref[...] * a + c
    hr_ref[...] = jnp.dot(h, wr_ref[...],
                          preferred_element_type=jnp.float32) + bgc_ref[...]
    m_ref[...] = jnp.dot(h, wn_ref[...], preferred_element_type=jnp.float32)


def _gc_body(hr_ref, a0_ref, a1_ref, w2_ref, b2_ref, t_ref, st_ref):
    i = pl.program_id(0)
    gc = hr_ref[...] + a0_ref[0] + a1_ref[0]
    g = gc * 0.5 * (1.0 + lax.erf(gc * 0.7071067811865476))
    t = jnp.dot(g, w2_ref[...], preferred_element_type=jnp.float32) + b2_ref[...]
    t_ref[...] = t

    @pl.when(i == 0)
    def _():
        st_ref[...] = jnp.zeros_like(st_ref)

    st_ref[0:1, :] += jnp.sum(t, axis=0, keepdims=True)
    st_ref[1:2, :] += jnp.sum(t * t, axis=0, keepdims=True)


def _fin_body(t_ref, st_ref, g_ref, be_ref, x_ref, o_ref):
    st = st_ref[...]
    mean = st[0:1, :] * (1.0 / N_NODES)
    var = st[1:2, :] * (1.0 / N_NODES) - mean * mean
    a = g_ref[...] * lax.rsqrt(var + EPS)
    c = be_ref[...] - mean * a
    o_ref[...] = t_ref[...] * a + c + x_ref[...]


_row_spec = pl.BlockSpec((_R, D), lambda i: (i, 0))
_full_spec = pl.BlockSpec((D, D), lambda i: (0, 0))
_vec_spec = pl.BlockSpec((1, D), lambda i: (0, 0))
_st_spec = pl.BlockSpec((8, D), lambda i: (0, 0))
_rows_out = jax.ShapeDtypeStruct((N_NODES, D), jnp.float32)
_st_out = jax.ShapeDtypeStruct((8, D), jnp.float32)


_fc1 = pl.pallas_call(
    _fc1_body, grid=(_G,),
    in_specs=[_row_spec, _full_spec, _vec_spec],
    out_specs=[_row_spec, _st_spec],
    out_shape=[_rows_out, _st_out],
)

_proj = pl.pallas_call(
    _proj_body, grid=(_G,),
    in_specs=[_row_spec, _st_spec, _vec_spec, _vec_spec, _full_spec,
              _full_spec, _vec_spec],
    out_specs=[_row_spec, _row_spec],
    out_shape=[_rows_out, _rows_out],
)

_gc = pl.pallas_call(
    _gc_body, grid=(_G,),
    in_specs=[_row_spec,
              pl.BlockSpec((1, _R, D), lambda i: (0, i, 0)),
              pl.BlockSpec((1, _R, D), lambda i: (1, i, 0)),
              _full_spec, _vec_spec],
    out_specs=[_row_spec, _st_spec],
    out_shape=[_rows_out, _st_out],
)

_fin = pl.pallas_call(
    _fin_body, grid=(_G,),
    in_specs=[_row_spec, _st_spec, _vec_spec, _vec_spec, _row_spec],
    out_specs=_row_spec,
    out_shape=_rows_out,
)


def _sc_body(m_hbm, src_hbm, dst_hbm, out_hbm,
             sidx, didx, rows, sidx_t, didx_t, rows_t, zbuf, agg_sh, gsem):
    cid = lax.axis_index("c")
    sid = lax.axis_index("s")
    base = cid * (_NS * _E_PER_TILE) + sid * _E_PER_TILE
    row0 = sid * _RPT

    z16 = jnp.zeros((16,), jnp.float32)

    def _zrow(r, carry):
        for j in range(D // 16):
            zbuf[r, pl.ds(j * 16, 16)] = z16
        return carry

    lax.fori_loop(0, _ZR, _zrow, 0)

    def _zcp(t, carry):
        pltpu.sync_copy(zbuf, agg_sh.at[pl.ds(row0 + t * _ZR, _ZR)])
        return carry

    lax.fori_loop(0, _RPT // _ZR, _zcp, 0)
    plsc.subcore_barrier()

    def _step(it, carry):
        e0 = base + it * _CH
        pltpu.sync_copy(src_hbm.at[pl.ds(e0, _CH)], sidx)
        pltpu.sync_copy(dst_hbm.at[pl.ds(e0, _CH)], didx)
        pltpu.async_copy(m_hbm.at[sidx], rows, gsem).wait()
        pltpu.sync_copy(rows, agg_sh.at[didx], add=True)
        return carry

    lax.fori_loop(0, _NFULL, _step, 0)

    e0 = base + _NFULL * _CH
    pltpu.sync_copy(src_hbm.at[pl.ds(e0, _TAIL)], sidx_t)
    pltpu.sync_copy(dst_hbm.at[pl.ds(e0, _TAIL)], didx_t)
    pltpu.async_copy(m_hbm.at[sidx_t], rows_t, gsem).wait()
    pltpu.sync_copy(rows_t, agg_sh.at[didx_t], add=True)

    plsc.subcore_barrier()

    def _wout(t, carry):
        r0 = row0 + t * _ZR
        pltpu.sync_copy(agg_sh.at[pl.ds(r0, _ZR)],
                        out_hbm.at[cid, pl.ds(r0, _ZR)])
        return carry

    lax.fori_loop(0, _RPT // _ZR, _wout, 0)


@functools.cache
def _make_segsum():
    return functools.partial(
        pl.kernel,
        mesh=plsc.VectorSubcoreMesh(core_axis_name="c", subcore_axis_name="s"),
        out_type=jax.ShapeDtypeStruct((_NC, _NPAD, D), jnp.float32),
        scratch_types=[
            pltpu.VMEM((_CH,), jnp.int32),
            pltpu.VMEM((_CH,), jnp.int32),
            pltpu.VMEM((_CH, D), jnp.float32),
            pltpu.VMEM((_TAIL,), jnp.int32),
            pltpu.VMEM((_TAIL,), jnp.int32),
            pltpu.VMEM((_TAIL, D), jnp.float32),
            pltpu.VMEM((_ZR, D), jnp.float32),
            pltpu.VMEM_SHARED((_NPAD, D), jnp.float32),
            pltpu.SemaphoreType.DMA,
        ],
    )(_sc_body)


def kernel(x, edge_index, W1, b1, g1, be1, Wr, Wn, bgc, W2, b2, g2, be2):
    ei = edge_index.astype(jnp.int32)
    src = ei[0]
    dst = ei[1]
    b1r = b1.reshape(1, D)
    g1r = g1.reshape(1, D)
    be1r = be1.reshape(1, D)
    bgcr = bgc.reshape(1, D)
    b2r = b2.reshape(1, D)
    g2r = g2.reshape(1, D)
    be2r = be2.reshape(1, D)

    h0, st1 = _fc1(x, W1, b1r)
    hr, m = _proj(h0, st1, g1r, be1r, Wr, Wn, bgcr)
    aggp = _make_segsum()(m, src, dst)
    t, st2 = _gc(hr, aggp, aggp, W2, b2r)
    return _fin(t, st2, g2r, be2r, x)


# speedup vs baseline: 3.1369x; 1.4583x over previous
import functools

import jax
import jax.numpy as jnp
from jax import lax
from jax.experimental import pallas as pl
from jax.experimental.pallas import tpu as pltpu
from jax.experimental.pallas import tpu_sc as plsc

N_NODES = 10000
D = 128
N_EDGES = 320000
EPS = 1e-5

_R = 2000
_G = N_NODES // _R

_NC = 2
_NS = 16
_E_PER_TILE = N_EDGES // (_NC * _NS)
_CH = 128
_NFULL = _E_PER_TILE // _CH
_TAIL = _E_PER_TILE - _NFULL * _CH
_NPAD = 10240
_RPT = _NPAD // _NS
_ZR = 128


def _fc1_body(x_ref, w_ref, b_ref, h0_ref, st_ref):
    i = pl.program_id(0)
    h0 = jnp.dot(x_ref[...], w_ref[...], preferred_element_type=jnp.float32)
    h0 = h0 + b_ref[...]
    h0_ref[...] = h0

    @pl.when(i == 0)
    def _():
        st_ref[...] = jnp.zeros_like(st_ref)

    st_ref[0:1, :] += jnp.sum(h0, axis=0, keepdims=True)
    st_ref[1:2, :] += jnp.sum(h0 * h0, axis=0, keepdims=True)


def _proj_body(h0_ref, st_ref, g_ref, be_ref, wr_ref, wn_ref, bgc_ref,
               hr_ref, m_ref):
    st = st_ref[...]
    mean = st[0:1, :] * (1.0 / N_NODES)
    var = st[1:2, :] * (1.0 / N_NODES) - mean * mean
    a = g_ref[...] * lax.rsqrt(var + EPS)
    c = be_ref[...] - mean * a
    h = h0_ref[...] * a + c
    hr_ref[...] = jnp.dot(h, wr_ref[...],
                          preferred_element_type=jnp.float32) + bgc_ref[...]
    m_ref[...] = jnp.dot(h, wn_ref[...], preferred_element_type=jnp.float32)


def _gc_body(hr_ref, a0_ref, a1_ref, w2_ref, b2_ref, t_ref, st_ref):
    i = pl.program_id(0)
    gc = hr_ref[...] + a0_ref[0] + a1_ref[0]
    g = gc * 0.5 * (1.0 + lax.erf(gc * 0.7071067811865476))
    t = jnp.dot(g, w2_ref[...], preferred_element_type=jnp.float32) + b2_ref[...]
    t_ref[...] = t

    @pl.when(i == 0)
    def _():
        st_ref[...] = jnp.zeros_like(st_ref)

    st_ref[0:1, :] += jnp.sum(t, axis=0, keepdims=True)
    st_ref[1:2, :] += jnp.sum(t * t, axis=0, keepdims=True)


def _fin_body(t_ref, st_ref, g_ref, be_ref, x_ref, o_ref):
    st = st_ref[...]
    mean = st[0:1, :] * (1.0 / N_NODES)
    var = st[1:2, :] * (1.0 / N_NODES) - mean * mean
    a = g_ref[...] * lax.rsqrt(var + EPS)
    c = be_ref[...] - mean * a
    o_ref[...] = t_ref[...] * a + c + x_ref[...]


_row_spec = pl.BlockSpec((_R, D), lambda i: (i, 0))
_full_spec = pl.BlockSpec((D, D), lambda i: (0, 0))
_vec_spec = pl.BlockSpec((1, D), lambda i: (0, 0))
_st_spec = pl.BlockSpec((8, D), lambda i: (0, 0))
_rows_out = jax.ShapeDtypeStruct((N_NODES, D), jnp.float32)
_st_out = jax.ShapeDtypeStruct((8, D), jnp.float32)


_fc1 = pl.pallas_call(
    _fc1_body, grid=(_G,),
    in_specs=[_row_spec, _full_spec, _vec_spec],
    out_specs=[_row_spec, _st_spec],
    out_shape=[_rows_out, _st_out],
)

_proj = pl.pallas_call(
    _proj_body, grid=(_G,),
    in_specs=[_row_spec, _st_spec, _vec_spec, _vec_spec, _full_spec,
              _full_spec, _vec_spec],
    out_specs=[_row_spec, _row_spec],
    out_shape=[_rows_out, _rows_out],
)

_gc = pl.pallas_call(
    _gc_body, grid=(_G,),
    in_specs=[_row_spec,
              pl.BlockSpec((1, _R, D), lambda i: (0, i, 0)),
              pl.BlockSpec((1, _R, D), lambda i: (1, i, 0)),
              _full_spec, _vec_spec],
    out_specs=[_row_spec, _st_spec],
    out_shape=[_rows_out, _st_out],
)

_fin = pl.pallas_call(
    _fin_body, grid=(_G,),
    in_specs=[_row_spec, _st_spec, _vec_spec, _vec_spec, _row_spec],
    out_specs=_row_spec,
    out_shape=_rows_out,
)


def _sc_body(m_hbm, src_hbm, dst_hbm, out_hbm,
             sidx, didx, sidx1, didx1, rows, sidx_t, didx_t, rows_t, zbuf,
             agg_sh, gsem, gsem1):
    cid = lax.axis_index("c")
    sid = lax.axis_index("s")
    base = cid * (_NS * _E_PER_TILE) + sid * _E_PER_TILE
    row0 = sid * _RPT

    z16 = jnp.zeros((16,), jnp.float32)

    def _zrow(r, carry):
        for j in range(D // 16):
            zbuf[r, pl.ds(j * 16, 16)] = z16
        return carry

    lax.fori_loop(0, _ZR, _zrow, 0)

    def _zcp(t, carry):
        pltpu.sync_copy(zbuf, agg_sh.at[pl.ds(row0 + t * _ZR, _ZR)])
        return carry

    lax.fori_loop(0, _RPT // _ZR, _zcp, 0)
    plsc.subcore_barrier()

    def _load(c, s_ref, d_ref):
        eoff = base + c * _CH
        pltpu.sync_copy(src_hbm.at[pl.ds(eoff, _CH)], s_ref)
        pltpu.sync_copy(dst_hbm.at[pl.ds(eoff, _CH)], d_ref)

    def _g(s_ref, r_ref, sem):
        return pltpu.make_async_copy(m_hbm.at[s_ref], r_ref, sem)

    _load(0, sidx, didx)
    _g(sidx, rows, gsem).start()

    def _pair(p, carry):
        c = 2 * p
        _load(c + 1, sidx1, didx1)
        _g(sidx1, zbuf, gsem1).start()
        _g(sidx, rows, gsem).wait()
        pltpu.sync_copy(rows, agg_sh.at[didx], add=True)

        @pl.when(c + 2 < _NFULL)
        def _():
            _load(c + 2, sidx, didx)
            _g(sidx, rows, gsem).start()

        _g(sidx1, zbuf, gsem1).wait()
        pltpu.sync_copy(zbuf, agg_sh.at[didx1], add=True)
        return carry

    lax.fori_loop(0, _NFULL // 2, _pair, 0)

    e0 = base + _NFULL * _CH
    pltpu.sync_copy(src_hbm.at[pl.ds(e0, _TAIL)], sidx_t)
    pltpu.sync_copy(dst_hbm.at[pl.ds(e0, _TAIL)], didx_t)
    pltpu.async_copy(m_hbm.at[sidx_t], rows_t, gsem).wait()
    pltpu.sync_copy(rows_t, agg_sh.at[didx_t], add=True)

    plsc.subcore_barrier()

    def _wout(t, carry):
        r0 = row0 + t * _ZR
        pltpu.sync_copy(agg_sh.at[pl.ds(r0, _ZR)],
                        out_hbm.at[cid, pl.ds(r0, _ZR)])
        return carry

    lax.fori_loop(0, _RPT // _ZR, _wout, 0)


@functools.cache
def _make_segsum():
    return functools.partial(
        pl.kernel,
        mesh=plsc.VectorSubcoreMesh(core_axis_name="c", subcore_axis_name="s"),
        out_type=jax.ShapeDtypeStruct((_NC, _NPAD, D), jnp.float32),
        scratch_types=[
            pltpu.VMEM((_CH,), jnp.int32),
            pltpu.VMEM((_CH,), jnp.int32),
            pltpu.VMEM((_CH,), jnp.int32),
            pltpu.VMEM((_CH,), jnp.int32),
            pltpu.VMEM((_CH, D), jnp.float32),
            pltpu.VMEM((_TAIL,), jnp.int32),
            pltpu.VMEM((_TAIL,), jnp.int32),
            pltpu.VMEM((_TAIL, D), jnp.float32),
            pltpu.VMEM((_ZR, D), jnp.float32),
            pltpu.VMEM_SHARED((_NPAD, D), jnp.float32),
            pltpu.SemaphoreType.DMA,
            pltpu.SemaphoreType.DMA,
        ],
    )(_sc_body)


def kernel(x, edge_index, W1, b1, g1, be1, Wr, Wn, bgc, W2, b2, g2, be2):
    ei = edge_index.astype(jnp.int32)
    src = ei[0]
    dst = ei[1]
    b1r = b1.reshape(1, D)
    g1r = g1.reshape(1, D)
    be1r = be1.reshape(1, D)
    bgcr = bgc.reshape(1, D)
    b2r = b2.reshape(1, D)
    g2r = g2.reshape(1, D)
    be2r = be2.reshape(1, D)

    h0, st1 = _fc1(x, W1, b1r)
    hr, m = _proj(h0, st1, g1r, be1r, Wr, Wn, bgcr)
    aggp = _make_segsum()(m, src, dst)
    t, st2 = _gc(hr, aggp, aggp, W2, b2r)
    return _fin(t, st2, g2r, be2r, x)
